# baseline (speedup 1.0000x reference)
import functools

import jax
import jax.numpy as jnp
from jax import lax
from jax.experimental import pallas as pl
from jax.experimental.pallas import tpu as pltpu
from jax.experimental.pallas import tpu_sc as plsc

VOCAB = 6
NAUX = 5
HID = 512
NC = 2
NS = 16
NW = NC * NS
CH = 64


def _body(ids_hbm, aux16_hbm, zeros_hbm, out_hbm, idsbuf, abuf, buf):
    n = out_hbm.shape[0]
    rows_per_w = n // NW
    nchunk = rows_per_w // CH
    wid = lax.axis_index("s") * NC + lax.axis_index("c")
    base0 = wid * rows_per_w

    pltpu.sync_copy(zeros_hbm, buf)

    iota = lax.iota(jnp.int32, 16)
    ones = jnp.ones((16,), jnp.float32)

    def chunk(c, carry):
        base = base0 + c * CH
        pltpu.sync_copy(ids_hbm.at[pl.ds(base, CH)], idsbuf)
        pltpu.sync_copy(aux16_hbm.at[pl.ds(base, CH)], abuf)
        for row in range(CH):
            buf[row, 0:16] = abuf[row]
        for g in range(CH // 16):
            rows16 = g * 16 + iota
            idsv = plsc.load_gather(idsbuf, [rows16])
            plsc.store_scatter(buf, [rows16, idsv], ones)
        pltpu.sync_copy(buf, out_hbm.at[pl.ds(base, CH)])
        return carry

    lax.fori_loop(0, nchunk, chunk, 0)


def kernel(input_ids, aux_features):
    B, S = input_ids.shape
    N = B * S
    ids1 = input_ids.reshape(N).astype(jnp.int32)
    aux16 = jnp.pad(aux_features.reshape(N, NAUX), ((0, 0), (VOCAB, 16 - VOCAB - NAUX)))
    zeros = jnp.zeros((CH, HID), jnp.float32)

    k = functools.partial(
        pl.kernel,
        out_type=jax.ShapeDtypeStruct((N, HID), jnp.float32),
        mesh=plsc.VectorSubcoreMesh(core_axis_name="c", subcore_axis_name="s"),
        compiler_params=pltpu.CompilerParams(needs_layout_passes=False),
        scratch_types=[
            pltpu.VMEM((CH,), jnp.int32),
            pltpu.VMEM((CH, 16), jnp.float32),
            pltpu.VMEM((CH, HID), jnp.float32),
        ],
    )(_body)
    out = k(ids1, aux16, zeros)
    return out.reshape(B, S, HID)
